# Initial kernel scaffold; baseline (speedup 1.0000x reference)
import jax
import jax.numpy as jnp
from jax.experimental import pallas as pl


def kernel(x_user, x_item, ei_user_to_item, ei_item_to_user, W_lin_user, b_lin_user, W_lin_item, b_lin_item, W_rel_u2i_0, b_rel_u2i_0, W_root_u2i_0, W_rel_i2u_0, b_rel_i2u_0, W_root_i2u_0, W_rel_u2i_1, b_rel_u2i_1, W_root_u2i_1, W_rel_i2u_1, b_rel_i2u_1, W_root_i2u_1, W_rel_u2i_2, b_rel_u2i_2, W_root_u2i_2, W_rel_i2u_2, b_rel_i2u_2, W_root_i2u_2, W_out, b_out):
    raise NotImplementedError("write your pallas kernel here")


# speedup vs baseline: 2.0734x; 2.0734x over previous
import functools

import jax
import jax.numpy as jnp
from jax import lax
from jax.experimental import pallas as pl
from jax.experimental.pallas import tpu as pltpu
from jax.experimental.pallas import tpu_sc as plsc

N = 50000
TILES = 16
ROWS_PER_TILE = 3136
NP = TILES * ROWS_PER_TILE
E = 500000
BLK = 512
JS = BLK // 128
NBLK = 62
EDGES_PER_TILE = NBLK * BLK
EP = TILES * EDGES_PER_TILE
EROWS_PER_TILE = EDGES_PER_TILE // 128
D = 128
QW = 32
DUMP_ROW = N

_f32 = jnp.float32
_i32 = jnp.int32


def _segsum_body(x4, srce, dste, out, srcb, dstb, gidx, rows, zbuf, acc,
                 gsem, ssem):
    c = lax.axis_index("c")
    s = lax.axis_index("s")

    zv = jnp.zeros((16,), _f32)

    def _zb(i, carry):
        zbuf[i, pl.ds(0, 16)] = zv
        zbuf[i, pl.ds(16, 16)] = zv
        return carry

    lax.fori_loop(0, 64, _zb, 0)

    row0 = s * ROWS_PER_TILE
    erow0 = s * EROWS_PER_TILE

    for qp in range(2):
        quarter = c * 2 + qp

        for k in range(ROWS_PER_TILE // 64):
            pltpu.sync_copy(zbuf, acc.at[pl.ds(row0 + k * 64, 64)])
        plsc.subcore_barrier()

        def _blk(b, carry):
            r0 = erow0 + b * JS
            pltpu.sync_copy(srce.at[pl.ds(r0, JS)], srcb)
            pltpu.sync_copy(dste.at[pl.ds(r0, JS)], dstb)
            for j in range(JS):
                for v in range(8):
                    sl = pl.ds(v * 16, 16)
                    gidx[j, sl] = srcb[j, sl] * 4 + quarter
            hs = [pltpu.async_copy(x4.at[gidx.at[j]], rows.at[j], gsem)
                  for j in range(JS)]
            for h in hs:
                h.wait()
            hs = [pltpu.async_copy(rows.at[j], acc.at[dstb.at[j]], ssem,
                                   add=True)
                  for j in range(JS)]
            for h in hs:
                h.wait()
            return carry

        lax.fori_loop(0, NBLK, _blk, 0)
        plsc.subcore_barrier()

        pltpu.sync_copy(acc.at[pl.ds(row0, ROWS_PER_TILE)],
                        out.at[quarter, pl.ds(row0, ROWS_PER_TILE)])
        plsc.subcore_barrier()


@jax.jit
def _segsum(x_pad, src2d, dst2d):
    x4 = x_pad.reshape(NP * 4, QW)
    mesh = plsc.VectorSubcoreMesh(core_axis_name="c", subcore_axis_name="s")
    f = pl.kernel(
        _segsum_body,
        out_type=jax.ShapeDtypeStruct((4, NP, QW), _f32),
        mesh=mesh,
        scratch_types=[
            pltpu.VMEM((JS, 128), _i32),
            pltpu.VMEM((JS, 128), _i32),
            pltpu.VMEM((JS, 128), _i32),
            pltpu.VMEM((JS, 128, QW), _f32),
            pltpu.VMEM((64, QW), _f32),
            pltpu.VMEM_SHARED((NP, QW), _f32),
            pltpu.SemaphoreType.DMA,
            pltpu.SemaphoreType.DMA,
        ],
        compiler_params=pltpu.CompilerParams(use_tc_tiling_on_sc=False),
    )
    return f(x4, src2d, dst2d)


_GRID = NP // ROWS_PER_TILE


def _lin_body(x_ref, w_ref, b_ref, o_ref, *, act):
    y = lax.dot_general(x_ref[...], w_ref[...], (((1,), (1,)), ((), ())),
                        preferred_element_type=_f32)
    y = y + b_ref[...]
    o_ref[...] = jnp.maximum(y, 0.0) if act else y


def _lin(x, w, b2, act):
    return pl.pallas_call(
        functools.partial(_lin_body, act=act),
        grid=(_GRID,),
        in_specs=[
            pl.BlockSpec((ROWS_PER_TILE, D), lambda i: (i, 0)),
            pl.BlockSpec((D, D), lambda i: (0, 0)),
            pl.BlockSpec((1, D), lambda i: (0, 0)),
        ],
        out_specs=pl.BlockSpec((ROWS_PER_TILE, D), lambda i: (i, 0)),
        out_shape=jax.ShapeDtypeStruct((x.shape[0], D), _f32),
    )(x, w, b2)


def _combine_body(a4_ref, x_ref, wrel_ref, b_ref, wroot_ref, o_ref):
    y = lax.dot_general(x_ref[...], wroot_ref[...], (((1,), (1,)), ((), ())),
                        preferred_element_type=_f32)
    for q in range(4):
        wq = wrel_ref[:, q * QW:(q + 1) * QW]
        y = y + lax.dot_general(a4_ref[q], wq, (((1,), (1,)), ((), ())),
                                preferred_element_type=_f32)
    o_ref[...] = jnp.maximum(y + b_ref[...], 0.0)


def _combine(a4, x, wrel, b2, wroot):
    return pl.pallas_call(
        _combine_body,
        grid=(_GRID,),
        in_specs=[
            pl.BlockSpec((4, ROWS_PER_TILE, QW), lambda i: (0, i, 0)),
            pl.BlockSpec((ROWS_PER_TILE, D), lambda i: (i, 0)),
            pl.BlockSpec((D, D), lambda i: (0, 0)),
            pl.BlockSpec((1, D), lambda i: (0, 0)),
            pl.BlockSpec((D, D), lambda i: (0, 0)),
        ],
        out_specs=pl.BlockSpec((ROWS_PER_TILE, D), lambda i: (i, 0)),
        out_shape=jax.ShapeDtypeStruct((NP, D), _f32),
    )(a4, x, wrel, b2, wroot)


def _prep_edges(ei):
    src = jnp.concatenate([ei[0], jnp.zeros((EP - E,), _i32)])
    dst = jnp.concatenate([ei[1], jnp.full((EP - E,), DUMP_ROW, _i32)])
    return src.reshape(EP // 128, 128), dst.reshape(EP // 128, 128)


def kernel(x_user, x_item, ei_user_to_item, ei_item_to_user, W_lin_user,
           b_lin_user, W_lin_item, b_lin_item, W_rel_u2i_0, b_rel_u2i_0,
           W_root_u2i_0, W_rel_i2u_0, b_rel_i2u_0, W_root_i2u_0, W_rel_u2i_1,
           b_rel_u2i_1, W_root_u2i_1, W_rel_i2u_1, b_rel_i2u_1, W_root_i2u_1,
           W_rel_u2i_2, b_rel_u2i_2, W_root_u2i_2, W_rel_i2u_2, b_rel_i2u_2,
           W_root_i2u_2, W_out, b_out):
    pad = lambda x: jnp.pad(x, ((0, NP - N), (0, 0)))
    b2 = lambda b: b.reshape(1, D)
    xu = pad(x_user)
    xi = pad(x_item)
    src_ui, dst_ui = _prep_edges(ei_user_to_item)
    src_iu, dst_iu = _prep_edges(ei_item_to_user)

    h_u = _lin(xu, W_lin_user, b2(b_lin_user), act=True)
    h_i = _lin(xi, W_lin_item, b2(b_lin_item), act=True)

    rel_u2i = (W_rel_u2i_0, W_rel_u2i_1, W_rel_u2i_2)
    brel_u2i = (b_rel_u2i_0, b_rel_u2i_1, b_rel_u2i_2)
    root_u2i = (W_root_u2i_0, W_root_u2i_1, W_root_u2i_2)
    rel_i2u = (W_rel_i2u_0, W_rel_i2u_1, W_rel_i2u_2)
    brel_i2u = (b_rel_i2u_0, b_rel_i2u_1, b_rel_i2u_2)
    root_i2u = (W_root_i2u_0, W_root_i2u_1, W_root_i2u_2)

    for l in range(3):
        agg_i4 = _segsum(h_u, src_ui, dst_ui)
        agg_u4 = _segsum(h_i, src_iu, dst_iu)
        new_i = _combine(agg_i4, h_i, rel_u2i[l], b2(brel_u2i[l]), root_u2i[l])
        new_u = _combine(agg_u4, h_u, rel_i2u[l], b2(brel_i2u[l]), root_i2u[l])
        h_u, h_i = new_u, new_i

    y_u = _lin(h_u, W_out, b2(b_out), act=False)[:N]
    y_i = _lin(h_i, W_out, b2(b_out), act=False)[:N]
    return (y_u, y_i)


# speedup vs baseline: 2.4743x; 1.1934x over previous
import functools

import jax
import jax.numpy as jnp
from jax import lax
from jax.experimental import pallas as pl
from jax.experimental.pallas import tpu as pltpu
from jax.experimental.pallas import tpu_sc as plsc

N = 50000
TILES = 16
ROWS_PER_TILE = 3136
NP = TILES * ROWS_PER_TILE
E = 500000
BLK = 256
JS = BLK // 128
NBLK = 124
EDGES_PER_TILE = NBLK * BLK
EP = TILES * EDGES_PER_TILE
EROWS_PER_TILE = EDGES_PER_TILE // 128
D = 128
QW = 32
DUMP_ROW = N

_f32 = jnp.float32
_i32 = jnp.int32


def _segsum_body(x4, edges, out, eb0, eb1, gi0, gi1, db0, db1, rw0, rw1,
                 zbuf, acc, es0, es1, gs0, gs1, ss0, ss1):
    c = lax.axis_index("c")
    s = lax.axis_index("s")
    ebuf = (eb0, eb1)
    gidx = (gi0, gi1)
    dstb = (db0, db1)
    rows = (rw0, rw1)
    esem = (es0, es1)
    gsem = (gs0, gs1)
    ssem = (ss0, ss1)

    zv = jnp.zeros((16,), _f32)

    def _zb(i, carry):
        zbuf[i, pl.ds(0, 16)] = zv
        zbuf[i, pl.ds(16, 16)] = zv
        return carry

    lax.fori_loop(0, 64, _zb, 0)

    row0 = s * ROWS_PER_TILE
    erow0 = s * EROWS_PER_TILE

    def _fire_idx(b, par):
        return pltpu.async_copy(edges.at[pl.ds(erow0 + b * JS, JS)],
                                ebuf[par], esem[par])

    def _fire_gather(par, quarter):
        for j in range(JS):
            for v in range(8):
                sl = pl.ds(v * 16, 16)
                gidx[par][j, sl] = ebuf[par][j, 0, sl] * 4 + quarter
                dstb[par][j, sl] = ebuf[par][j, 1, sl]
        return [pltpu.async_copy(x4.at[gidx[par].at[j]], rows[par].at[j],
                                 gsem[par])
                for j in range(JS)]

    def _scatter(par):
        hs = [pltpu.async_copy(rows[par].at[j], acc.at[dstb[par].at[j]],
                               ssem[par], add=True)
              for j in range(JS)]
        for h in hs:
            h.wait()

    for qp in range(2):
        quarter = c * 2 + qp

        h_idx0 = _fire_idx(0, 0)
        for k in range(ROWS_PER_TILE // 64):
            pltpu.sync_copy(zbuf, acc.at[pl.ds(row0 + k * 64, 64)])
        plsc.subcore_barrier()

        h_idx0.wait()
        g0 = _fire_gather(0, quarter)
        _fire_idx(1, 1).wait()

        def _pair(p, carry):
            b = p * 2
            _fire_gather(1, quarter)
            h2 = _fire_idx(b + 2, 0)
            _scatter_wait(0)
            h2.wait()
            _fire_gather(0, quarter)

            @pl.when(b + 3 < NBLK)
            def _():
                _fire_idx(b + 3, 1).wait()

            _scatter_wait(1)
            return carry

        def _scatter_wait(par):
            for j in range(JS):
                pltpu.make_async_copy(x4.at[gidx[par].at[j]],
                                      rows[par].at[j], gsem[par]).wait()
            _scatter(par)

        lax.fori_loop(0, NBLK // 2 - 1, _pair, 0)
        _fire_gather(1, quarter)
        _scatter_wait(0)
        _scatter_wait(1)
        plsc.subcore_barrier()

        pltpu.sync_copy(acc.at[pl.ds(row0, ROWS_PER_TILE)],
                        out.at[quarter, pl.ds(row0, ROWS_PER_TILE)])
        plsc.subcore_barrier()


@jax.jit
def _segsum(x_pad, edges):
    x4 = x_pad.reshape(NP * 4, QW)
    mesh = plsc.VectorSubcoreMesh(core_axis_name="c", subcore_axis_name="s")
    f = pl.kernel(
        _segsum_body,
        out_type=jax.ShapeDtypeStruct((4, NP, QW), _f32),
        mesh=mesh,
        scratch_types=(
            [pltpu.VMEM((JS, 2, 128), _i32)] * 2
            + [pltpu.VMEM((JS, 128), _i32)] * 2
            + [pltpu.VMEM((JS, 128), _i32)] * 2
            + [pltpu.VMEM((JS, 128, QW), _f32)] * 2
            + [pltpu.VMEM((64, QW), _f32)]
            + [pltpu.VMEM_SHARED((NP, QW), _f32)]
            + [pltpu.SemaphoreType.DMA] * 6
        ),
        compiler_params=pltpu.CompilerParams(use_tc_tiling_on_sc=False),
    )
    return f(x4, edges)


_GRID = NP // ROWS_PER_TILE


def _lin_body(x_ref, w_ref, b_ref, o_ref, *, act):
    y = lax.dot_general(x_ref[...], w_ref[...], (((1,), (1,)), ((), ())),
                        preferred_element_type=_f32)
    y = y + b_ref[...]
    o_ref[...] = jnp.maximum(y, 0.0) if act else y


def _lin(x, w, b2, act):
    return pl.pallas_call(
        functools.partial(_lin_body, act=act),
        grid=(_GRID,),
        in_specs=[
            pl.BlockSpec((ROWS_PER_TILE, D), lambda i: (i, 0)),
            pl.BlockSpec((D, D), lambda i: (0, 0)),
            pl.BlockSpec((1, D), lambda i: (0, 0)),
        ],
        out_specs=pl.BlockSpec((ROWS_PER_TILE, D), lambda i: (i, 0)),
        out_shape=jax.ShapeDtypeStruct((x.shape[0], D), _f32),
    )(x, w, b2)


def _combine_body(a4_ref, x_ref, wrel_ref, b_ref, wroot_ref, o_ref):
    y = lax.dot_general(x_ref[...], wroot_ref[...], (((1,), (1,)), ((), ())),
                        preferred_element_type=_f32)
    for q in range(4):
        wq = wrel_ref[:, q * QW:(q + 1) * QW]
        y = y + lax.dot_general(a4_ref[q], wq, (((1,), (1,)), ((), ())),
                                preferred_element_type=_f32)
    o_ref[...] = jnp.maximum(y + b_ref[...], 0.0)


def _combine(a4, x, wrel, b2, wroot):
    return pl.pallas_call(
        _combine_body,
        grid=(_GRID,),
        in_specs=[
            pl.BlockSpec((4, ROWS_PER_TILE, QW), lambda i: (0, i, 0)),
            pl.BlockSpec((ROWS_PER_TILE, D), lambda i: (i, 0)),
            pl.BlockSpec((D, D), lambda i: (0, 0)),
            pl.BlockSpec((1, D), lambda i: (0, 0)),
            pl.BlockSpec((D, D), lambda i: (0, 0)),
        ],
        out_specs=pl.BlockSpec((ROWS_PER_TILE, D), lambda i: (i, 0)),
        out_shape=jax.ShapeDtypeStruct((NP, D), _f32),
    )(a4, x, wrel, b2, wroot)


def _prep_edges(ei):
    src = jnp.concatenate([ei[0], jnp.zeros((EP - E,), _i32)])
    dst = jnp.concatenate([ei[1], jnp.full((EP - E,), DUMP_ROW, _i32)])
    return jnp.stack([src.reshape(EP // 128, 128),
                      dst.reshape(EP // 128, 128)], axis=1)


def kernel(x_user, x_item, ei_user_to_item, ei_item_to_user, W_lin_user,
           b_lin_user, W_lin_item, b_lin_item, W_rel_u2i_0, b_rel_u2i_0,
           W_root_u2i_0, W_rel_i2u_0, b_rel_i2u_0, W_root_i2u_0, W_rel_u2i_1,
           b_rel_u2i_1, W_root_u2i_1, W_rel_i2u_1, b_rel_i2u_1, W_root_i2u_1,
           W_rel_u2i_2, b_rel_u2i_2, W_root_u2i_2, W_rel_i2u_2, b_rel_i2u_2,
           W_root_i2u_2, W_out, b_out):
    pad = lambda x: jnp.pad(x, ((0, NP - N), (0, 0)))
    b2 = lambda b: b.reshape(1, D)
    xu = pad(x_user)
    xi = pad(x_item)
    ed_ui = _prep_edges(ei_user_to_item)
    ed_iu = _prep_edges(ei_item_to_user)

    h_u = _lin(xu, W_lin_user, b2(b_lin_user), act=True)
    h_i = _lin(xi, W_lin_item, b2(b_lin_item), act=True)

    rel_u2i = (W_rel_u2i_0, W_rel_u2i_1, W_rel_u2i_2)
    brel_u2i = (b_rel_u2i_0, b_rel_u2i_1, b_rel_u2i_2)
    root_u2i = (W_root_u2i_0, W_root_u2i_1, W_root_u2i_2)
    rel_i2u = (W_rel_i2u_0, W_rel_i2u_1, W_rel_i2u_2)
    brel_i2u = (b_rel_i2u_0, b_rel_i2u_1, b_rel_i2u_2)
    root_i2u = (W_root_i2u_0, W_root_i2u_1, W_root_i2u_2)

    for l in range(3):
        agg_i4 = _segsum(h_u, ed_ui)
        agg_u4 = _segsum(h_i, ed_iu)
        new_i = _combine(agg_i4, h_i, rel_u2i[l], b2(brel_u2i[l]), root_u2i[l])
        new_u = _combine(agg_u4, h_u, rel_i2u[l], b2(brel_i2u[l]), root_i2u[l])
        h_u, h_i = new_u, new_i

    y_u = _lin(h_u, W_out, b2(b_out), act=False)[:N]
    y_i = _lin(h_i, W_out, b2(b_out), act=False)[:N]
    return (y_u, y_i)


# speedup vs baseline: 3.3028x; 1.3348x over previous
import functools

import jax
import jax.numpy as jnp
from jax import lax
from jax.experimental import pallas as pl
from jax.experimental.pallas import tpu as pltpu
from jax.experimental.pallas import tpu_sc as plsc

N = 50000
TILES = 16
ROWS_PER_TILE = 3136
NP = TILES * ROWS_PER_TILE
E = 500000
BLK = 256
JS = BLK // 128
NBLK = 124
EDGES_PER_TILE = NBLK * BLK
EP = TILES * EDGES_PER_TILE
EROWS_PER_TILE = EDGES_PER_TILE // 128
D = 128
HW = 64
DUMP_ROW = N

_f32 = jnp.float32
_bf16 = jnp.bfloat16
_i32 = jnp.int32


def _segsum2_body(xa2, ea, xb2, eb, outa, outb, eb0, eb1, gi0, gi1, db0, db1,
                  rw0, rw1, zbuf, acc, es0, es1, gs0, gs1, ss0, ss1):
    c = lax.axis_index("c")
    s = lax.axis_index("s")
    ebuf = (eb0, eb1)
    gidx = (gi0, gi1)
    dstb = (db0, db1)
    rows = (rw0, rw1)
    esem = (es0, es1)
    gsem = (gs0, gs1)
    ssem = (ss0, ss1)

    zv = jnp.zeros((32,), _bf16)

    def _zb(i, carry):
        zbuf[i, pl.ds(0, 32)] = zv
        zbuf[i, pl.ds(32, 32)] = zv
        return carry

    lax.fori_loop(0, 64, _zb, 0)

    row0 = s * ROWS_PER_TILE
    erow0 = s * EROWS_PER_TILE

    for et in range(2):
        x2 = (xa2, xb2)[et]
        edges = (ea, eb)[et]
        out = (outa, outb)[et]

        def _fire_idx(b, par):
            return pltpu.async_copy(edges.at[pl.ds(erow0 + b * JS, JS)],
                                    ebuf[par], esem[par])

        def _fire_gather(par):
            for j in range(JS):
                for v in range(8):
                    sl = pl.ds(v * 16, 16)
                    gidx[par][j, sl] = ebuf[par][j, 0, sl] * 2 + c
                    dstb[par][j, sl] = ebuf[par][j, 1, sl]
            return [pltpu.async_copy(x2.at[gidx[par].at[j]],
                                     rows[par].at[j], gsem[par])
                    for j in range(JS)]

        def _scatter_wait(par):
            for j in range(JS):
                pltpu.make_async_copy(x2.at[gidx[par].at[j]],
                                      rows[par].at[j], gsem[par]).wait()
            hs = [pltpu.async_copy(rows[par].at[j], acc.at[dstb[par].at[j]],
                                   ssem[par], add=True)
                  for j in range(JS)]
            for h in hs:
                h.wait()

        h_idx0 = _fire_idx(0, 0)
        for k in range(ROWS_PER_TILE // 64):
            pltpu.sync_copy(zbuf, acc.at[pl.ds(row0 + k * 64, 64)])
        plsc.subcore_barrier()

        h_idx0.wait()
        _fire_gather(0)
        _fire_idx(1, 1).wait()

        def _pair(p, carry):
            b = p * 2
            _fire_gather(1)
            h2 = _fire_idx(b + 2, 0)
            _scatter_wait(0)
            h2.wait()
            _fire_gather(0)

            @pl.when(b + 3 < NBLK)
            def _():
                _fire_idx(b + 3, 1).wait()

            _scatter_wait(1)
            return carry

        lax.fori_loop(0, NBLK // 2 - 1, _pair, 0)
        _fire_gather(1)
        _scatter_wait(0)
        _scatter_wait(1)
        plsc.subcore_barrier()

        pltpu.sync_copy(acc.at[pl.ds(row0, ROWS_PER_TILE)],
                        out.at[c, pl.ds(row0, ROWS_PER_TILE)])
        plsc.subcore_barrier()


@jax.jit
def _segsum2(xa_bf, ea, xb_bf, eb):
    xa2 = xa_bf.reshape(NP * 2, HW)
    xb2 = xb_bf.reshape(NP * 2, HW)
    mesh = plsc.VectorSubcoreMesh(core_axis_name="c", subcore_axis_name="s")
    f = pl.kernel(
        _segsum2_body,
        out_type=(jax.ShapeDtypeStruct((2, NP, HW), _bf16),
                  jax.ShapeDtypeStruct((2, NP, HW), _bf16)),
        mesh=mesh,
        scratch_types=(
            [pltpu.VMEM((JS, 2, 128), _i32)] * 2
            + [pltpu.VMEM((JS, 128), _i32)] * 2
            + [pltpu.VMEM((JS, 128), _i32)] * 2
            + [pltpu.VMEM((JS, 128, HW), _bf16)] * 2
            + [pltpu.VMEM((64, HW), _bf16)]
            + [pltpu.VMEM_SHARED((NP, HW), _bf16)]
            + [pltpu.SemaphoreType.DMA] * 6
        ),
        compiler_params=pltpu.CompilerParams(use_tc_tiling_on_sc=False),
    )
    return f(xa2, ea, xb2, eb)


_GRID = NP // ROWS_PER_TILE


def _lin_body(x_ref, w_ref, b_ref, o_ref, obf_ref, *, act):
    y = lax.dot_general(x_ref[...], w_ref[...], (((1,), (1,)), ((), ())),
                        preferred_element_type=_f32)
    y = y + b_ref[...]
    if act:
        y = jnp.maximum(y, 0.0)
    o_ref[...] = y
    obf_ref[...] = y.astype(_bf16)


def _lin(x, w, b2, act):
    return pl.pallas_call(
        functools.partial(_lin_body, act=act),
        grid=(_GRID,),
        in_specs=[
            pl.BlockSpec((ROWS_PER_TILE, D), lambda i: (i, 0)),
            pl.BlockSpec((D, D), lambda i: (0, 0)),
            pl.BlockSpec((1, D), lambda i: (0, 0)),
        ],
        out_specs=[
            pl.BlockSpec((ROWS_PER_TILE, D), lambda i: (i, 0)),
            pl.BlockSpec((ROWS_PER_TILE, D), lambda i: (i, 0)),
        ],
        out_shape=[
            jax.ShapeDtypeStruct((x.shape[0], D), _f32),
            jax.ShapeDtypeStruct((x.shape[0], D), _bf16),
        ],
    )(x, w, b2)


def _combine_body(a2_ref, x_ref, wrel_ref, b_ref, wroot_ref, o_ref, obf_ref):
    y = lax.dot_general(x_ref[...], wroot_ref[...], (((1,), (1,)), ((), ())),
                        preferred_element_type=_f32)
    for h in range(2):
        wh = wrel_ref[:, h * HW:(h + 1) * HW]
        ah = a2_ref[h].astype(_f32)
        y = y + lax.dot_general(ah, wh, (((1,), (1,)), ((), ())),
                                preferred_element_type=_f32)
    y = jnp.maximum(y + b_ref[...], 0.0)
    o_ref[...] = y
    obf_ref[...] = y.astype(_bf16)


def _combine(a2, x, wrel, b2, wroot):
    return pl.pallas_call(
        _combine_body,
        grid=(_GRID,),
        in_specs=[
            pl.BlockSpec((2, ROWS_PER_TILE, HW), lambda i: (0, i, 0)),
            pl.BlockSpec((ROWS_PER_TILE, D), lambda i: (i, 0)),
            pl.BlockSpec((D, D), lambda i: (0, 0)),
            pl.BlockSpec((1, D), lambda i: (0, 0)),
            pl.BlockSpec((D, D), lambda i: (0, 0)),
        ],
        out_specs=[
            pl.BlockSpec((ROWS_PER_TILE, D), lambda i: (i, 0)),
            pl.BlockSpec((ROWS_PER_TILE, D), lambda i: (i, 0)),
        ],
        out_shape=[
            jax.ShapeDtypeStruct((NP, D), _f32),
            jax.ShapeDtypeStruct((NP, D), _bf16),
        ],
    )(a2, x, wrel, b2, wroot)


def _final_body(x_ref, w_ref, b_ref, o_ref):
    y = lax.dot_general(x_ref[...], w_ref[...], (((1,), (1,)), ((), ())),
                        preferred_element_type=_f32)
    o_ref[...] = y + b_ref[...]


def _final(x, w, b2):
    return pl.pallas_call(
        _final_body,
        grid=(_GRID,),
        in_specs=[
            pl.BlockSpec((ROWS_PER_TILE, D), lambda i: (i, 0)),
            pl.BlockSpec((D, D), lambda i: (0, 0)),
            pl.BlockSpec((1, D), lambda i: (0, 0)),
        ],
        out_specs=pl.BlockSpec((ROWS_PER_TILE, D), lambda i: (i, 0)),
        out_shape=jax.ShapeDtypeStruct((NP, D), _f32),
    )(x, w, b2)


def _prep_edges(ei):
    src = jnp.concatenate([ei[0], jnp.zeros((EP - E,), _i32)])
    dst = jnp.concatenate([ei[1], jnp.full((EP - E,), DUMP_ROW, _i32)])
    return jnp.stack([src.reshape(EP // 128, 128),
                      dst.reshape(EP // 128, 128)], axis=1)


def kernel(x_user, x_item, ei_user_to_item, ei_item_to_user, W_lin_user,
           b_lin_user, W_lin_item, b_lin_item, W_rel_u2i_0, b_rel_u2i_0,
           W_root_u2i_0, W_rel_i2u_0, b_rel_i2u_0, W_root_i2u_0, W_rel_u2i_1,
           b_rel_u2i_1, W_root_u2i_1, W_rel_i2u_1, b_rel_i2u_1, W_root_i2u_1,
           W_rel_u2i_2, b_rel_u2i_2, W_root_u2i_2, W_rel_i2u_2, b_rel_i2u_2,
           W_root_i2u_2, W_out, b_out):
    pad = lambda x: jnp.pad(x, ((0, NP - N), (0, 0)))
    b2 = lambda b: b.reshape(1, D)
    xu = pad(x_user)
    xi = pad(x_item)
    ed_ui = _prep_edges(ei_user_to_item)
    ed_iu = _prep_edges(ei_item_to_user)

    h_u, hbf_u = _lin(xu, W_lin_user, b2(b_lin_user), act=True)
    h_i, hbf_i = _lin(xi, W_lin_item, b2(b_lin_item), act=True)

    rel_u2i = (W_rel_u2i_0, W_rel_u2i_1, W_rel_u2i_2)
    brel_u2i = (b_rel_u2i_0, b_rel_u2i_1, b_rel_u2i_2)
    root_u2i = (W_root_u2i_0, W_root_u2i_1, W_root_u2i_2)
    rel_i2u = (W_rel_i2u_0, W_rel_i2u_1, W_rel_i2u_2)
    brel_i2u = (b_rel_i2u_0, b_rel_i2u_1, b_rel_i2u_2)
    root_i2u = (W_root_i2u_0, W_root_i2u_1, W_root_i2u_2)

    for l in range(3):
        agg_i2, agg_u2 = _segsum2(hbf_u, ed_ui, hbf_i, ed_iu)
        new_i, nbf_i = _combine(agg_i2, h_i, rel_u2i[l], b2(brel_u2i[l]),
                                root_u2i[l])
        new_u, nbf_u = _combine(agg_u2, h_u, rel_i2u[l], b2(brel_i2u[l]),
                                root_i2u[l])
        h_u, h_i = new_u, new_i
        hbf_u, hbf_i = nbf_u, nbf_i

    y_u = _final(h_u, W_out, b2(b_out))[:N]
    y_i = _final(h_i, W_out, b2(b_out))[:N]
    return (y_u, y_i)


# speedup vs baseline: 3.6368x; 1.1012x over previous
import jax
import jax.numpy as jnp
from jax import lax
from jax.experimental import pallas as pl
from jax.experimental.pallas import tpu as pltpu
from jax.experimental.pallas import tpu_sc as plsc

N = 50000
TILES = 16
ROWS_PER_TILE = 3136
NP = TILES * ROWS_PER_TILE
E = 500000
BLK = 256
JS = BLK // 128
NBLK = 124
EDGES_PER_TILE = NBLK * BLK
EP = TILES * EDGES_PER_TILE
EROWS_PER_TILE = EDGES_PER_TILE // 128
D = 128
HW = 64
DUMP_ROW = N

_f32 = jnp.float32
_bf16 = jnp.bfloat16
_i32 = jnp.int32


def _segsum2_body(x2, ea, eb, out, eb0, eb1, gi0, gi1, db0, db1,
                  rw0, rw1, zbuf, acc, es0, es1, gs0, gs1, ss0, ss1):
    c = lax.axis_index("c")
    s = lax.axis_index("s")
    ebuf = (eb0, eb1)
    gidx = (gi0, gi1)
    dstb = (db0, db1)
    rows = (rw0, rw1)
    esem = (es0, es1)
    gsem = (gs0, gs1)
    ssem = (ss0, ss1)

    zv = jnp.zeros((32,), _bf16)

    def _zb(i, carry):
        zbuf[i, pl.ds(0, 32)] = zv
        zbuf[i, pl.ds(32, 32)] = zv
        return carry

    lax.fori_loop(0, 64, _zb, 0)

    row0 = s * ROWS_PER_TILE
    erow0 = s * EROWS_PER_TILE

    for t in range(2):
        edges = (ea, eb)[t]
        base = (1 - t) * (NP * 2) + c

        def _fire_idx(b, par):
            return pltpu.async_copy(edges.at[pl.ds(erow0 + b * JS, JS)],
                                    ebuf[par], esem[par])

        def _fire_gather(par):
            for j in range(JS):
                for v in range(8):
                    sl = pl.ds(v * 16, 16)
                    gidx[par][j, sl] = ebuf[par][j, 0, sl] * 2 + base
                    dstb[par][j, sl] = ebuf[par][j, 1, sl]
            return [pltpu.async_copy(x2.at[gidx[par].at[j]],
                                     rows[par].at[j], gsem[par])
                    for j in range(JS)]

        def _scatter_wait(par):
            for j in range(JS):
                pltpu.make_async_copy(x2.at[gidx[par].at[j]],
                                      rows[par].at[j], gsem[par]).wait()
            hs = [pltpu.async_copy(rows[par].at[j], acc.at[dstb[par].at[j]],
                                   ssem[par], add=True)
                  for j in range(JS)]
            for h in hs:
                h.wait()

        h_idx0 = _fire_idx(0, 0)
        for k in range(ROWS_PER_TILE // 64):
            pltpu.sync_copy(zbuf, acc.at[pl.ds(row0 + k * 64, 64)])
        plsc.subcore_barrier()

        h_idx0.wait()
        _fire_gather(0)
        _fire_idx(1, 1).wait()

        def _pair(p, carry):
            b = p * 2
            _fire_gather(1)
            h2 = _fire_idx(b + 2, 0)
            _scatter_wait(0)
            h2.wait()
            _fire_gather(0)

            @pl.when(b + 3 < NBLK)
            def _():
                _fire_idx(b + 3, 1).wait()

            _scatter_wait(1)
            return carry

        lax.fori_loop(0, NBLK // 2 - 1, _pair, 0)
        _fire_gather(1)
        _scatter_wait(0)
        _scatter_wait(1)
        plsc.subcore_barrier()

        pltpu.sync_copy(acc.at[pl.ds(row0, ROWS_PER_TILE)],
                        out.at[t, c, pl.ds(row0, ROWS_PER_TILE)])
        plsc.subcore_barrier()


@jax.jit
def _segsum2(hb, ea, eb):
    x2 = hb.reshape(2 * NP * 2, HW)
    mesh = plsc.VectorSubcoreMesh(core_axis_name="c", subcore_axis_name="s")
    f = pl.kernel(
        _segsum2_body,
        out_type=jax.ShapeDtypeStruct((2, 2, NP, HW), _bf16),
        mesh=mesh,
        scratch_types=(
            [pltpu.VMEM((JS, 2, 128), _i32)] * 2
            + [pltpu.VMEM((JS, 128), _i32)] * 2
            + [pltpu.VMEM((JS, 128), _i32)] * 2
            + [pltpu.VMEM((JS, 128, HW), _bf16)] * 2
            + [pltpu.VMEM((64, HW), _bf16)]
            + [pltpu.VMEM_SHARED((NP, HW), _bf16)]
            + [pltpu.SemaphoreType.DMA] * 6
        ),
        compiler_params=pltpu.CompilerParams(use_tc_tiling_on_sc=False),
    )
    return f(x2, ea, eb)


_GRID = NP // ROWS_PER_TILE
_CT = (((1,), (1,)), ((), ()))


def _init_body(xu_ref, xi_ref, w_ref, b_ref, hb_ref):
    for t in range(2):
        x = (xu_ref, xi_ref)[t][...]
        y = lax.dot_general(x, w_ref[t], _CT, preferred_element_type=_f32)
        hb_ref[t] = jnp.maximum(y + b_ref[t], 0.0).astype(_bf16)


def _init(x_user, x_item, w_st, b_st):
    return pl.pallas_call(
        _init_body,
        grid=(_GRID,),
        in_specs=[
            pl.BlockSpec((ROWS_PER_TILE, D), lambda i: (i, 0)),
            pl.BlockSpec((ROWS_PER_TILE, D), lambda i: (i, 0)),
            pl.BlockSpec((2, D, D), lambda i: (0, 0, 0)),
            pl.BlockSpec((2, 1, D), lambda i: (0, 0, 0)),
        ],
        out_specs=pl.BlockSpec((2, ROWS_PER_TILE, D), lambda i: (0, i, 0)),
        out_shape=jax.ShapeDtypeStruct((2, NP, D), _bf16),
    )(x_user, x_item, w_st, b_st)


def _combine_body(a_ref, hb_ref, wrel_ref, b_ref, wroot_ref, o_ref):
    for t in range(2):
        y = lax.dot_general(hb_ref[t], wroot_ref[t], _CT,
                            preferred_element_type=_f32)
        for h in range(2):
            wh = wrel_ref[t, :, h * HW:(h + 1) * HW]
            y = y + lax.dot_general(a_ref[t, h], wh, _CT,
                                    preferred_element_type=_f32)
        o_ref[t] = jnp.maximum(y + b_ref[t], 0.0).astype(_bf16)


def _combine(a, hb, wrel_st, b_st, wroot_st):
    return pl.pallas_call(
        _combine_body,
        grid=(_GRID,),
        in_specs=[
            pl.BlockSpec((2, 2, ROWS_PER_TILE, HW), lambda i: (0, 0, i, 0)),
            pl.BlockSpec((2, ROWS_PER_TILE, D), lambda i: (0, i, 0)),
            pl.BlockSpec((2, D, D), lambda i: (0, 0, 0)),
            pl.BlockSpec((2, 1, D), lambda i: (0, 0, 0)),
            pl.BlockSpec((2, D, D), lambda i: (0, 0, 0)),
        ],
        out_specs=pl.BlockSpec((2, ROWS_PER_TILE, D), lambda i: (0, i, 0)),
        out_shape=jax.ShapeDtypeStruct((2, NP, D), _bf16),
    )(a, hb, wrel_st, b_st, wroot_st)


def _final_body(hb_ref, w_ref, b_ref, ou_ref, oi_ref):
    for t in range(2):
        y = lax.dot_general(hb_ref[t], w_ref[...], _CT,
                            preferred_element_type=_f32)
        (ou_ref, oi_ref)[t][...] = y + b_ref[...]


def _final(hb, w, b2):
    return pl.pallas_call(
        _final_body,
        grid=(_GRID,),
        in_specs=[
            pl.BlockSpec((2, ROWS_PER_TILE, D), lambda i: (0, i, 0)),
            pl.BlockSpec((D, D), lambda i: (0, 0)),
            pl.BlockSpec((1, D), lambda i: (0, 0)),
        ],
        out_specs=[
            pl.BlockSpec((ROWS_PER_TILE, D), lambda i: (i, 0)),
            pl.BlockSpec((ROWS_PER_TILE, D), lambda i: (i, 0)),
        ],
        out_shape=[
            jax.ShapeDtypeStruct((N, D), _f32),
            jax.ShapeDtypeStruct((N, D), _f32),
        ],
    )(hb, w, b2)


def _prep_edges(ei):
    src = jnp.concatenate([ei[0], jnp.zeros((EP - E,), _i32)])
    dst = jnp.concatenate([ei[1], jnp.full((EP - E,), DUMP_ROW, _i32)])
    return jnp.stack([src.reshape(EP // 128, 128),
                      dst.reshape(EP // 128, 128)], axis=1)


def kernel(x_user, x_item, ei_user_to_item, ei_item_to_user, W_lin_user,
           b_lin_user, W_lin_item, b_lin_item, W_rel_u2i_0, b_rel_u2i_0,
           W_root_u2i_0, W_rel_i2u_0, b_rel_i2u_0, W_root_i2u_0, W_rel_u2i_1,
           b_rel_u2i_1, W_root_u2i_1, W_rel_i2u_1, b_rel_i2u_1, W_root_i2u_1,
           W_rel_u2i_2, b_rel_u2i_2, W_root_u2i_2, W_rel_i2u_2, b_rel_i2u_2,
           W_root_i2u_2, W_out, b_out):
    ed_iu = _prep_edges(ei_item_to_user)
    ed_ui = _prep_edges(ei_user_to_item)

    hb = _init(x_user, x_item,
               jnp.stack([W_lin_user, W_lin_item]),
               jnp.stack([b_lin_user.reshape(1, D),
                          b_lin_item.reshape(1, D)]))

    rel = ((W_rel_i2u_0, W_rel_u2i_0), (W_rel_i2u_1, W_rel_u2i_1),
           (W_rel_i2u_2, W_rel_u2i_2))
    brel = ((b_rel_i2u_0, b_rel_u2i_0), (b_rel_i2u_1, b_rel_u2i_1),
            (b_rel_i2u_2, b_rel_u2i_2))
    root = ((W_root_i2u_0, W_root_u2i_0), (W_root_i2u_1, W_root_u2i_1),
            (W_root_i2u_2, W_root_u2i_2))

    for l in range(3):
        a = _segsum2(hb, ed_iu, ed_ui)
        hb = _combine(a, hb,
                      jnp.stack(rel[l]),
                      jnp.stack([brel[l][0].reshape(1, D),
                                 brel[l][1].reshape(1, D)]),
                      jnp.stack(root[l]))

    y_u, y_i = _final(hb, W_out, b_out.reshape(1, D))
    return (y_u, y_i)


# speedup vs baseline: 3.9246x; 1.0791x over previous
import jax
import jax.numpy as jnp
from jax import lax
from jax.experimental import pallas as pl
from jax.experimental.pallas import tpu as pltpu
from jax.experimental.pallas import tpu_sc as plsc

N = 50000
TILES = 16
ROWS_PER_TILE = 3136
NP = TILES * ROWS_PER_TILE
E = 500000
BLK = 256
JS = BLK // 128
NBLK = 124
EDGES_PER_TILE = NBLK * BLK
EP = TILES * EDGES_PER_TILE
EROWS_PER_TILE = EDGES_PER_TILE // 128
D = 128
HW = 64
DUMP_ROW = N

_f32 = jnp.float32
_bf16 = jnp.bfloat16
_i32 = jnp.int32


def _segsum_body(x2, edges, out, eb0, eb1, gi0, gi1, db0, db1,
                 rw0, rw1, zbuf, acc, es0, es1, gs0, gs1, ss0, ss1):
    c = lax.axis_index("c")
    s = lax.axis_index("s")
    ebuf = (eb0, eb1)
    gidx = (gi0, gi1)
    dstb = (db0, db1)
    rows = (rw0, rw1)
    esem = (es0, es1)
    gsem = (gs0, gs1)
    ssem = (ss0, ss1)

    zv = jnp.zeros((32,), _bf16)

    def _zb(i, carry):
        zbuf[i, pl.ds(0, 32)] = zv
        zbuf[i, pl.ds(32, 32)] = zv
        return carry

    lax.fori_loop(0, 64, _zb, 0)

    row0 = s * ROWS_PER_TILE
    erow0 = s * EROWS_PER_TILE

    if True:
        base = c

        def _fire_idx(b, par):
            return pltpu.async_copy(edges.at[pl.ds(erow0 + b * JS, JS)],
                                    ebuf[par], esem[par])

        def _fire_gather(par):
            for j in range(JS):
                for v in range(8):
                    sl = pl.ds(v * 16, 16)
                    gidx[par][j, sl] = ebuf[par][j, 0, sl] * 2 + base
                    dstb[par][j, sl] = ebuf[par][j, 1, sl]
            return [pltpu.async_copy(x2.at[gidx[par].at[j]],
                                     rows[par].at[j], gsem[par])
                    for j in range(JS)]

        def _scatter_wait(par):
            for j in range(JS):
                pltpu.make_async_copy(x2.at[gidx[par].at[j]],
                                      rows[par].at[j], gsem[par]).wait()
            hs = [pltpu.async_copy(rows[par].at[j], acc.at[dstb[par].at[j]],
                                   ssem[par], add=True)
                  for j in range(JS)]
            for h in hs:
                h.wait()

        h_idx0 = _fire_idx(0, 0)
        for k in range(ROWS_PER_TILE // 64):
            pltpu.sync_copy(zbuf, acc.at[pl.ds(row0 + k * 64, 64)])
        plsc.subcore_barrier()

        h_idx0.wait()
        _fire_gather(0)
        _fire_idx(1, 1).wait()

        def _pair(p, carry):
            b = p * 2
            _fire_gather(1)
            h2 = _fire_idx(b + 2, 0)
            _scatter_wait(0)
            h2.wait()
            _fire_gather(0)

            @pl.when(b + 3 < NBLK)
            def _():
                _fire_idx(b + 3, 1).wait()

            _scatter_wait(1)
            return carry

        lax.fori_loop(0, NBLK // 2 - 1, _pair, 0)
        _fire_gather(1)
        _scatter_wait(0)
        _scatter_wait(1)
        plsc.subcore_barrier()

        pltpu.sync_copy(acc.at[pl.ds(row0, ROWS_PER_TILE)],
                        out.at[c, pl.ds(row0, ROWS_PER_TILE)])
        plsc.subcore_barrier()


@jax.jit
def _segsum(h_src, edges):
    x2 = h_src.reshape(NP * 2, HW)
    mesh = plsc.VectorSubcoreMesh(core_axis_name="c", subcore_axis_name="s")
    f = pl.kernel(
        _segsum_body,
        out_type=jax.ShapeDtypeStruct((2, NP, HW), _bf16),
        mesh=mesh,
        scratch_types=(
            [pltpu.VMEM((JS, 2, 128), _i32)] * 2
            + [pltpu.VMEM((JS, 128), _i32)] * 2
            + [pltpu.VMEM((JS, 128), _i32)] * 2
            + [pltpu.VMEM((JS, 128, HW), _bf16)] * 2
            + [pltpu.VMEM((64, HW), _bf16)]
            + [pltpu.VMEM_SHARED((NP, HW), _bf16)]
            + [pltpu.SemaphoreType.DMA] * 6
        ),
        compiler_params=pltpu.CompilerParams(use_tc_tiling_on_sc=False),
    )
    return f(x2, edges)


_GRID = NP // ROWS_PER_TILE
_CT = (((1,), (1,)), ((), ()))


def _init_body(xu_ref, xi_ref, w_ref, b_ref, hu_ref, hi_ref):
    for t in range(2):
        x = (xu_ref, xi_ref)[t][...]
        y = lax.dot_general(x, w_ref[t], _CT, preferred_element_type=_f32)
        (hu_ref, hi_ref)[t][...] = jnp.maximum(y + b_ref[t], 0.0).astype(_bf16)


def _init(x_user, x_item, w_st, b_st):
    return pl.pallas_call(
        _init_body,
        grid=(_GRID,),
        in_specs=[
            pl.BlockSpec((ROWS_PER_TILE, D), lambda i: (i, 0)),
            pl.BlockSpec((ROWS_PER_TILE, D), lambda i: (i, 0)),
            pl.BlockSpec((2, D, D), lambda i: (0, 0, 0)),
            pl.BlockSpec((2, 1, D), lambda i: (0, 0, 0)),
        ],
        out_specs=[
            pl.BlockSpec((ROWS_PER_TILE, D), lambda i: (i, 0)),
            pl.BlockSpec((ROWS_PER_TILE, D), lambda i: (i, 0)),
        ],
        out_shape=[
            jax.ShapeDtypeStruct((NP, D), _bf16),
            jax.ShapeDtypeStruct((NP, D), _bf16),
        ],
    )(x_user, x_item, w_st, b_st)


def _combine_body(a_ref, h_ref, wrel_ref, b_ref, wroot_ref, o_ref):
    y = lax.dot_general(h_ref[...], wroot_ref[...], _CT,
                        preferred_element_type=_f32)
    for h in range(2):
        wh = wrel_ref[:, h * HW:(h + 1) * HW]
        y = y + lax.dot_general(a_ref[h], wh, _CT,
                                preferred_element_type=_f32)
    o_ref[...] = jnp.maximum(y + b_ref[...], 0.0).astype(_bf16)


def _combine(a, h, wrel, b2, wroot):
    return pl.pallas_call(
        _combine_body,
        grid=(_GRID,),
        in_specs=[
            pl.BlockSpec((2, ROWS_PER_TILE, HW), lambda i: (0, i, 0)),
            pl.BlockSpec((ROWS_PER_TILE, D), lambda i: (i, 0)),
            pl.BlockSpec((D, D), lambda i: (0, 0)),
            pl.BlockSpec((1, D), lambda i: (0, 0)),
            pl.BlockSpec((D, D), lambda i: (0, 0)),
        ],
        out_specs=pl.BlockSpec((ROWS_PER_TILE, D), lambda i: (i, 0)),
        out_shape=jax.ShapeDtypeStruct((NP, D), _bf16),
    )(a, h, wrel, b2, wroot)


def _final_body(hu_ref, hi_ref, w_ref, b_ref, ou_ref, oi_ref):
    for t in range(2):
        y = lax.dot_general((hu_ref, hi_ref)[t][...], w_ref[...], _CT,
                            preferred_element_type=_f32)
        (ou_ref, oi_ref)[t][...] = y + b_ref[...]


def _final(hu, hi, w, b2):
    return pl.pallas_call(
        _final_body,
        grid=(_GRID,),
        in_specs=[
            pl.BlockSpec((ROWS_PER_TILE, D), lambda i: (i, 0)),
            pl.BlockSpec((ROWS_PER_TILE, D), lambda i: (i, 0)),
            pl.BlockSpec((D, D), lambda i: (0, 0)),
            pl.BlockSpec((1, D), lambda i: (0, 0)),
        ],
        out_specs=[
            pl.BlockSpec((ROWS_PER_TILE, D), lambda i: (i, 0)),
            pl.BlockSpec((ROWS_PER_TILE, D), lambda i: (i, 0)),
        ],
        out_shape=[
            jax.ShapeDtypeStruct((N, D), _f32),
            jax.ShapeDtypeStruct((N, D), _f32),
        ],
    )(hu, hi, w, b2)


def _prep_edges(ei):
    src = jnp.concatenate([ei[0], jnp.zeros((EP - E,), _i32)])
    dst = jnp.concatenate([ei[1], jnp.full((EP - E,), DUMP_ROW, _i32)])
    return jnp.stack([src.reshape(EP // 128, 128),
                      dst.reshape(EP // 128, 128)], axis=1)


def kernel(x_user, x_item, ei_user_to_item, ei_item_to_user, W_lin_user,
           b_lin_user, W_lin_item, b_lin_item, W_rel_u2i_0, b_rel_u2i_0,
           W_root_u2i_0, W_rel_i2u_0, b_rel_i2u_0, W_root_i2u_0, W_rel_u2i_1,
           b_rel_u2i_1, W_root_u2i_1, W_rel_i2u_1, b_rel_i2u_1, W_root_i2u_1,
           W_rel_u2i_2, b_rel_u2i_2, W_root_u2i_2, W_rel_i2u_2, b_rel_i2u_2,
           W_root_i2u_2, W_out, b_out):
    ed_iu = _prep_edges(ei_item_to_user)
    ed_ui = _prep_edges(ei_user_to_item)

    h_u, h_i = _init(x_user, x_item,
                     jnp.stack([W_lin_user, W_lin_item]),
                     jnp.stack([b_lin_user.reshape(1, D),
                                b_lin_item.reshape(1, D)]))

    rel_u2i = (W_rel_u2i_0, W_rel_u2i_1, W_rel_u2i_2)
    brel_u2i = (b_rel_u2i_0, b_rel_u2i_1, b_rel_u2i_2)
    root_u2i = (W_root_u2i_0, W_root_u2i_1, W_root_u2i_2)
    rel_i2u = (W_rel_i2u_0, W_rel_i2u_1, W_rel_i2u_2)
    brel_i2u = (b_rel_i2u_0, b_rel_i2u_1, b_rel_i2u_2)
    root_i2u = (W_root_i2u_0, W_root_i2u_1, W_root_i2u_2)

    for l in range(3):
        agg_i = _segsum(h_u, ed_ui)
        agg_u = _segsum(h_i, ed_iu)
        new_i = _combine(agg_i, h_i, rel_u2i[l],
                         brel_u2i[l].reshape(1, D), root_u2i[l])
        new_u = _combine(agg_u, h_u, rel_i2u[l],
                         brel_i2u[l].reshape(1, D), root_i2u[l])
        h_u, h_i = new_u, new_i

    y_u, y_i = _final(h_u, h_i, W_out, b_out.reshape(1, D))
    return (y_u, y_i)
